# baseline (speedup 1.0000x reference)
import jax
import jax.numpy as jnp
from jax import lax
from jax.experimental import pallas as pl
from jax.experimental.pallas import tpu as pltpu, tpu_sc as plsc

RHO = 70.0
T, N = 1025, 20000
NPOS = 65536
SPAIRS = 100
TM1 = T - 1
NW = 32
PP = NPOS // NW
RT = TM1 // NW
SS = RT * SPAIRS
L = 16
SSP = SS + L
NPAD = 20096
NVR = (SPAIRS + L - 1) // L


def _sigmoid(z):
    return 1.0 / (1.0 + jnp.exp(-z))


def _body(x_hbm, th_hbm, tp_hbm, up_hbm, vp_hbm, us_hbm, vs_hbm,
          kpos_hbm, kneg_hbm,
          th_v, tp_v, up_v, vp_v, iu_v, iv_v, gu_v, gv_v,
          su_v, sv_v, rb0, rb1, op_v, on_v, psem, rsem):
    wid = lax.axis_index("s") * 2 + lax.axis_index("c")

    pltpu.sync_copy(th_hbm, th_v)
    eps = _sigmoid(th_v[...]) * 0.5

    base = wid * PP
    pltpu.sync_copy(tp_hbm.at[pl.ds(base, PP)], tp_v)
    pltpu.sync_copy(up_hbm.at[pl.ds(base, PP)], up_v)
    pltpu.sync_copy(vp_hbm.at[pl.ds(base, PP)], vp_v)

    def pos_idx(k, c):
        sl = pl.ds(k * L, L)
        roff = tp_v[sl] * N
        iu_v[sl] = roff + up_v[sl]
        iv_v[sl] = roff + vp_v[sl]
        return c
    lax.fori_loop(0, PP // L, pos_idx, 0)

    cu = pltpu.async_copy(x_hbm.at[iu_v], gu_v, psem)
    cv = pltpu.async_copy(x_hbm.at[iv_v], gv_v, psem)

    pltpu.sync_copy(us_hbm.at[pl.ds(wid * SS, SS)], su_v.at[pl.ds(0, SS)])
    pltpu.sync_copy(vs_hbm.at[pl.ds(wid * SS, SS)], sv_v.at[pl.ds(0, SS)])

    row0 = wid * RT
    pltpu.async_copy(x_hbm.at[pl.ds(row0 * N, N)], rb0.at[pl.ds(0, N)], rsem)
    pltpu.async_copy(x_hbm.at[pl.ds((row0 + 1) * N, N)], rb1.at[pl.ds(0, N)], rsem)

    iota = lax.iota(jnp.int32, L)

    def row_kappa(rb, rr):
        def one_vec(q, acc):
            sl = pl.ds(rr * SPAIRS + q * L, L)
            cu_ = jnp.minimum(jnp.maximum(su_v[sl], 0), N - 1)
            cv_ = jnp.minimum(jnp.maximum(sv_v[sl], 0), N - 1)
            xu = plsc.load_gather(rb, [cu_])
            xv = plsc.load_gather(rb, [cv_])
            kap = _sigmoid(RHO * (eps - jnp.abs(xu - xv)))
            nvalid = SPAIRS - q * L
            kap = jnp.where(iota < nvalid, kap, 0.0)
            return acc + kap
        acc = jnp.zeros((L,), jnp.float32)
        acc = lax.fori_loop(0, NVR, one_vec, acc)
        return jnp.sum(acc)

    def pair_body(i, carry):
        on_a, on_b = carry
        r0 = 2 * i
        r1 = 2 * i + 1

        pltpu.make_async_copy(x_hbm.at[pl.ds(0, N)], rb0.at[pl.ds(0, N)], rsem).wait()
        s0 = row_kappa(rb0, r0)
        nxt0 = row0 + jnp.minimum(r0 + 2, RT - 1)
        pltpu.async_copy(x_hbm.at[pl.ds(nxt0 * N, N)], rb0.at[pl.ds(0, N)], rsem)

        pltpu.make_async_copy(x_hbm.at[pl.ds(0, N)], rb1.at[pl.ds(0, N)], rsem).wait()
        s1 = row_kappa(rb1, r1)
        nxt1 = row0 + jnp.minimum(r1 + 2, RT - 1)
        pltpu.async_copy(x_hbm.at[pl.ds(nxt1 * N, N)], rb1.at[pl.ds(0, N)], rsem)

        k0 = jnp.where(iota == (r0 & (L - 1)), 1.0 - s0 * (1.0 / SPAIRS), 0.0)
        k1 = jnp.where(iota == (r1 & (L - 1)), 1.0 - s1 * (1.0 / SPAIRS), 0.0)
        both = k0 + k1
        in_a = jnp.where(r0 < L, both, 0.0)
        return (on_a + in_a, on_b + (both - in_a))

    zero = jnp.zeros((L,), jnp.float32)
    on_a, on_b = lax.fori_loop(0, RT // 2, pair_body, (zero, zero))
    pltpu.make_async_copy(x_hbm.at[pl.ds(0, N)], rb0.at[pl.ds(0, N)], rsem).wait()
    pltpu.make_async_copy(x_hbm.at[pl.ds(0, N)], rb1.at[pl.ds(0, N)], rsem).wait()

    on_v[pl.ds(0, L)] = on_a
    on_v[pl.ds(L, L)] = on_b
    pltpu.sync_copy(on_v, kneg_hbm.at[pl.ds(wid * RT, RT)])

    cu.wait()
    cv.wait()

    def pos_kap(k, c):
        sl = pl.ds(k * L, L)
        d = gu_v[sl] - gv_v[sl]
        op_v[sl] = _sigmoid(RHO * (eps - jnp.abs(d)))
        return c
    lax.fori_loop(0, PP // L, pos_kap, 0)
    pltpu.sync_copy(op_v, kpos_hbm.at[pl.ds(base, PP)])


def kernel(X, theta, u_pos, v_pos, t_pos, u_sample, v_sample):
    x_flat = X.reshape(-1)
    th16 = jnp.broadcast_to(theta.astype(jnp.float32), (L,))
    us_f = u_sample.reshape(-1)
    vs_f = v_sample.reshape(-1)

    mesh = plsc.VectorSubcoreMesh(core_axis_name="c", subcore_axis_name="s")
    run = pl.kernel(
        _body,
        out_type=(
            jax.ShapeDtypeStruct((NPOS,), jnp.float32),
            jax.ShapeDtypeStruct((TM1,), jnp.float32),
        ),
        mesh=mesh,
        compiler_params=pltpu.CompilerParams(
            use_tc_tiling_on_sc=False, needs_layout_passes=False),
        scratch_types=[
            pltpu.VMEM((L,), jnp.float32),
            pltpu.VMEM((PP,), jnp.int32),
            pltpu.VMEM((PP,), jnp.int32),
            pltpu.VMEM((PP,), jnp.int32),
            pltpu.VMEM((PP,), jnp.int32),
            pltpu.VMEM((PP,), jnp.int32),
            pltpu.VMEM((PP,), jnp.float32),
            pltpu.VMEM((PP,), jnp.float32),
            pltpu.VMEM((SSP,), jnp.int32),
            pltpu.VMEM((SSP,), jnp.int32),
            pltpu.VMEM((NPAD,), jnp.float32),
            pltpu.VMEM((NPAD,), jnp.float32),
            pltpu.VMEM((PP,), jnp.float32),
            pltpu.VMEM((RT,), jnp.float32),
            pltpu.SemaphoreType.DMA,
            pltpu.SemaphoreType.DMA,
        ],
    )
    kappa_pos, kappa_neg = run(x_flat, th16, t_pos, u_pos, v_pos, us_f, vs_f)
    return kappa_pos, kappa_neg
